# baseline (speedup 1.0000x reference)
import functools

import jax
import jax.numpy as jnp
import numpy as np
from jax import lax
from jax.experimental import pallas as pl
from jax.experimental.pallas import tpu as pltpu
from jax.experimental.pallas import tpu_sc as plsc

D_MODEL = 128
MAX_LEN = 512

_NUM_CORES = 2
_NUM_SUBCORES = 16
_NUM_WORKERS = _NUM_CORES * _NUM_SUBCORES

_CHUNK = 128


def _position_indices(shapes):
    D = shapes[0][1] - 1
    idx_list = []
    for s in shapes:
        t = s[1] - 1
        indices = np.concatenate(
            [np.zeros([1]), np.linspace(1, D, t).astype(np.int32)]
        )
        idx_list.append(indices.astype(np.int64))
    return np.concatenate(idx_list, axis=0).astype(np.int32)


def _make_sc_gather(total: int):
    n_main = total // _CHUNK
    tail = total - n_main * _CHUNK
    tail_pad = -tail % 8 if tail else 0
    tail_n = tail + tail_pad
    total_pad = n_main * _CHUNK + tail_n

    mesh = plsc.VectorSubcoreMesh(
        core_axis_name="c", subcore_axis_name="s",
        num_cores=_NUM_CORES, num_subcores=_NUM_SUBCORES,
    )

    scratch = [
        pltpu.VMEM((max(n_main, 1), _CHUNK), jnp.int32),
        pltpu.VMEM((max(tail_n, 8),), jnp.int32),
        pltpu.VMEM((total_pad, D_MODEL), jnp.float32),
        pltpu.SemaphoreType.DMA,
    ]

    @functools.partial(
        pl.kernel,
        out_type=jax.ShapeDtypeStruct(
            (_NUM_WORKERS, total, D_MODEL), jnp.float32
        ),
        mesh=mesh,
        scratch_types=scratch,
    )
    def sc_kernel(idx_main_hbm, idx_tail_hbm, emb_hbm, out_hbm,
                  idx_main_v, idx_tail_v, rows_v, sem):
        wid = lax.axis_index("s") * _NUM_CORES + lax.axis_index("c")
        pltpu.sync_copy(idx_main_hbm, idx_main_v)
        if tail:
            pltpu.sync_copy(idx_tail_hbm, idx_tail_v)
        copies = []
        for c in range(n_main):
            copies.append(pltpu.async_copy(
                emb_hbm.at[idx_main_v.at[c]],
                rows_v.at[pl.ds(c * _CHUNK, _CHUNK)],
                sem,
            ))
        if tail:
            copies.append(pltpu.async_copy(
                emb_hbm.at[idx_tail_v],
                rows_v.at[pl.ds(n_main * _CHUNK, tail_n)],
                sem,
            ))
        for cp in copies:
            cp.wait()
        pltpu.sync_copy(rows_v.at[pl.ds(0, total)], out_hbm.at[wid])

    return sc_kernel


def kernel(modal_feat_0, modal_feat_1, modal_feat_2, emb):
    shapes = [modal_feat_0.shape, modal_feat_1.shape, modal_feat_2.shape]
    idx = _position_indices(shapes)
    total = idx.shape[0]

    n_main = total // _CHUNK
    tail = total - n_main * _CHUNK
    tail_n = tail + (-tail % 8) if tail else 8

    idx_main = idx[: n_main * _CHUNK].reshape(max(n_main, 1), _CHUNK)
    tail_vals = np.zeros((tail_n,), np.int32)
    if tail:
        tail_vals[:tail] = idx[n_main * _CHUNK:]

    sc_gather = _make_sc_gather(total)
    return sc_gather(
        jnp.asarray(idx_main), jnp.asarray(tail_vals), emb
    )


# speedup vs baseline: 1.0761x; 1.0761x over previous
import functools

import jax
import jax.numpy as jnp
import numpy as np
from jax import lax
from jax.experimental import pallas as pl
from jax.experimental.pallas import tpu as pltpu
from jax.experimental.pallas import tpu_sc as plsc

D_MODEL = 128
MAX_LEN = 512

_NUM_CORES = 2
_NUM_SUBCORES = 16
_NUM_WORKERS = _NUM_CORES * _NUM_SUBCORES

_CHUNK = 128


def _position_indices(shapes):
    D = shapes[0][1] - 1
    idx_list = []
    for s in shapes:
        t = s[1] - 1
        indices = np.concatenate(
            [np.zeros([1]), np.linspace(1, D, t).astype(np.int32)]
        )
        idx_list.append(indices.astype(np.int64))
    return np.concatenate(idx_list, axis=0).astype(np.int32)


def _make_sc_gather(total: int, batch: int):
    stride = -(-total // _NUM_WORKERS)
    stride = stride + (-stride % 8)
    n_main = total // stride
    tail = total - n_main * stride
    rows_pad = stride + tail + (-(stride + tail) % 8)
    n_tail_workers = _NUM_WORKERS - n_main
    per_tail = -(-batch // n_tail_workers) if (tail and n_tail_workers) else 0

    mesh = plsc.VectorSubcoreMesh(
        core_axis_name="c", subcore_axis_name="s",
        num_cores=_NUM_CORES, num_subcores=_NUM_SUBCORES,
    )

    scratch = [
        pltpu.VMEM((rows_pad,), jnp.int32),
        pltpu.VMEM((rows_pad, D_MODEL), jnp.float32),
        pltpu.SemaphoreType.DMA,
        pltpu.SemaphoreType.DMA,
    ]

    @functools.partial(
        pl.kernel,
        out_type=jax.ShapeDtypeStruct((batch, total, D_MODEL), jnp.float32),
        mesh=mesh,
        scratch_types=scratch,
    )
    def sc_kernel(idx_hbm, emb_hbm, out_hbm, idx_v, rows_v, gsem, ssem):
        wid = lax.axis_index("s") * _NUM_CORES + lax.axis_index("c")
        base = pl.multiple_of(
            jnp.minimum(wid, n_main - 1) * stride, stride)
        pltpu.sync_copy(idx_hbm.at[wid], idx_v)
        pltpu.async_copy(emb_hbm.at[idx_v], rows_v, gsem).wait()

        @pl.when(wid < n_main)
        def _main():
            copies = []
            for b in range(batch):
                copies.append(pltpu.async_copy(
                    rows_v.at[pl.ds(0, stride)],
                    out_hbm.at[b, pl.ds(base, stride)],
                    ssem,
                ))
            for cp in copies:
                cp.wait()

        if tail:
            @pl.when(wid >= n_main)
            def _tail():
                tb = (wid - n_main) * per_tail
                copies = []
                for i in range(per_tail):
                    b = jnp.minimum(tb + i, batch - 1)
                    copies.append(pltpu.async_copy(
                        rows_v.at[pl.ds(stride, tail)],
                        out_hbm.at[b, pl.ds(n_main * stride, tail)],
                        ssem,
                    ))
                for cp in copies:
                    cp.wait()

    return sc_kernel


def kernel(modal_feat_0, modal_feat_1, modal_feat_2, emb):
    shapes = [modal_feat_0.shape, modal_feat_1.shape, modal_feat_2.shape]
    batch = shapes[0][0]
    idx = _position_indices(shapes)
    total = idx.shape[0]

    stride = -(-total // _NUM_WORKERS)
    stride = stride + (-stride % 8)
    n_main = total // stride
    tail = total - n_main * stride
    rows_pad = stride + tail + (-(stride + tail) % 8)
    assert tail == 0 or n_main < _NUM_WORKERS
    idx_chunks = np.zeros((_NUM_WORKERS, rows_pad), np.int32)
    for w in range(_NUM_WORKERS):
        base = min(w, n_main - 1) * stride
        idx_chunks[w, :stride] = idx[base: base + stride]
        if tail:
            idx_chunks[w, stride: stride + tail] = idx[n_main * stride:]

    sc_gather = _make_sc_gather(total, batch)
    return sc_gather(jnp.asarray(idx_chunks), emb)


# speedup vs baseline: 2.0313x; 1.8876x over previous
import functools

import jax
import jax.numpy as jnp
import numpy as np
from jax.experimental import pallas as pl
from jax.experimental.pallas import tpu as pltpu

D_MODEL = 128
MAX_LEN = 512


def _position_indices(shapes):
    D = shapes[0][1] - 1
    idx_list = []
    for s in shapes:
        t = s[1] - 1
        indices = np.concatenate(
            [np.zeros([1]), np.linspace(1, D, t).astype(np.int32)]
        )
        idx_list.append(indices.astype(np.int64))
    return np.concatenate(idx_list, axis=0).astype(np.int32)


def _make_fused_gather_broadcast(total: int, batch: int, vocab: int):
    total_pad = total + (-total % 8)

    def body(idx_ref, emb_ref, out_ref, g_ref):
        @pl.when(pl.program_id(0) == 0)
        def _gather():
            cols = jax.lax.broadcasted_iota(jnp.int32, (total_pad, vocab), 1)
            onehot = (cols == idx_ref[...]).astype(jnp.float32)
            g_ref[...] = jnp.dot(
                onehot, emb_ref[...],
                preferred_element_type=jnp.float32,
                precision=jax.lax.Precision.HIGHEST,
            )

        out_ref[0] = g_ref[pl.ds(0, total)]

    return pl.pallas_call(
        body,
        grid=(batch,),
        in_specs=[
            pl.BlockSpec((total_pad, 1), lambda b: (0, 0)),
            pl.BlockSpec((vocab, D_MODEL), lambda b: (0, 0)),
        ],
        out_specs=pl.BlockSpec((1, total, D_MODEL), lambda b: (b, 0, 0)),
        out_shape=jax.ShapeDtypeStruct((batch, total, D_MODEL), jnp.float32),
        scratch_shapes=[pltpu.VMEM((total_pad, D_MODEL), jnp.float32)],
    )


def kernel(modal_feat_0, modal_feat_1, modal_feat_2, emb):
    shapes = [modal_feat_0.shape, modal_feat_1.shape, modal_feat_2.shape]
    batch = shapes[0][0]
    idx = _position_indices(shapes)
    total = idx.shape[0]
    total_pad = total + (-total % 8)

    idx_col = np.zeros((total_pad, 1), np.int32)
    idx_col[:total, 0] = idx

    fused = _make_fused_gather_broadcast(total, batch, emb.shape[0])
    return fused(jnp.asarray(idx_col), emb)


# speedup vs baseline: 6.4684x; 3.1844x over previous
import functools

import jax
import jax.numpy as jnp
import numpy as np
from jax.experimental import pallas as pl
from jax.experimental.pallas import tpu as pltpu

D_MODEL = 128
MAX_LEN = 512


def _position_indices(shapes):
    D = shapes[0][1] - 1
    idx_list = []
    for s in shapes:
        t = s[1] - 1
        indices = np.concatenate(
            [np.zeros([1]), np.linspace(1, D, t).astype(np.int32)]
        )
        idx_list.append(indices.astype(np.int64))
    return np.concatenate(idx_list, axis=0).astype(np.int32)


_R_BLK = 128


def _make_fused_gather_broadcast(total: int, batch: int, vocab: int):
    n_steps = -(-total // _R_BLK)

    def body(idx_ref, emb_ref, out_ref):
        cols = jax.lax.broadcasted_iota(jnp.int32, (_R_BLK, vocab), 1)
        onehot = (cols == idx_ref[...]).astype(jnp.float32)
        rows = jnp.dot(
            onehot, emb_ref[...],
            preferred_element_type=jnp.float32,
            precision=jax.lax.Precision.HIGHEST,
        )
        out_ref[...] = jnp.broadcast_to(
            rows[:, None, :], (_R_BLK, batch, D_MODEL)
        )

    return pl.pallas_call(
        body,
        grid=(n_steps,),
        in_specs=[
            pl.BlockSpec((_R_BLK, 1), lambda r: (r, 0)),
            pl.BlockSpec((vocab, D_MODEL), lambda r: (0, 0)),
        ],
        out_specs=pl.BlockSpec((_R_BLK, batch, D_MODEL), lambda r: (r, 0, 0)),
        out_shape=jax.ShapeDtypeStruct((total, batch, D_MODEL), jnp.float32),
    )


def kernel(modal_feat_0, modal_feat_1, modal_feat_2, emb):
    shapes = [modal_feat_0.shape, modal_feat_1.shape, modal_feat_2.shape]
    batch = shapes[0][0]
    idx = _position_indices(shapes)
    total = idx.shape[0]

    n_steps = -(-total // _R_BLK)
    idx_col = np.zeros((n_steps * _R_BLK, 1), np.int32)
    idx_col[:total, 0] = idx

    fused = _make_fused_gather_broadcast(total, batch, emb.shape[0])
    out_t = fused(jnp.asarray(idx_col), emb)
    return jnp.transpose(out_t, (1, 0, 2))


# speedup vs baseline: 7.5863x; 1.1728x over previous
import functools

import jax
import jax.numpy as jnp
import numpy as np
from jax.experimental import pallas as pl
from jax.experimental.pallas import tpu as pltpu

D_MODEL = 128
MAX_LEN = 512


def _position_indices(shapes):
    D = shapes[0][1] - 1
    idx_list = []
    for s in shapes:
        t = s[1] - 1
        indices = np.concatenate(
            [np.zeros([1]), np.linspace(1, D, t).astype(np.int32)]
        )
        idx_list.append(indices.astype(np.int64))
    return np.concatenate(idx_list, axis=0).astype(np.int32)


_R_BLK = 256


def _make_fused_gather_broadcast(total: int, batch: int, vocab: int):
    n_steps = -(-total // _R_BLK)

    def body(idx_ref, emb_ref, out_ref):
        cols = jax.lax.broadcasted_iota(jnp.int32, (_R_BLK, vocab), 1)
        onehot = (cols == idx_ref[...]).astype(jnp.float32)
        rows = jnp.dot(
            onehot, emb_ref[...],
            preferred_element_type=jnp.float32,
            precision=jax.lax.Precision.HIGHEST,
        )
        out_ref[...] = jnp.broadcast_to(
            rows[:, None, :], (_R_BLK, batch, D_MODEL)
        )

    return pl.pallas_call(
        body,
        grid=(n_steps,),
        in_specs=[
            pl.BlockSpec((_R_BLK, 1), lambda r: (r, 0)),
            pl.BlockSpec((vocab, D_MODEL), lambda r: (0, 0)),
        ],
        out_specs=pl.BlockSpec((_R_BLK, batch, D_MODEL), lambda r: (r, 0, 0)),
        out_shape=jax.ShapeDtypeStruct((total, batch, D_MODEL), jnp.float32),
    )


def kernel(modal_feat_0, modal_feat_1, modal_feat_2, emb):
    shapes = [modal_feat_0.shape, modal_feat_1.shape, modal_feat_2.shape]
    batch = shapes[0][0]
    idx = _position_indices(shapes)
    total = idx.shape[0]

    n_steps = -(-total // _R_BLK)
    idx_col = np.zeros((n_steps * _R_BLK, 1), np.int32)
    idx_col[:total, 0] = idx

    fused = _make_fused_gather_broadcast(total, batch, emb.shape[0])
    out_t = fused(jnp.asarray(idx_col), emb)
    return jnp.transpose(out_t, (1, 0, 2))


# speedup vs baseline: 7.8827x; 1.0391x over previous
import functools

import jax
import jax.numpy as jnp
import numpy as np
from jax.experimental import pallas as pl
from jax.experimental.pallas import tpu as pltpu

D_MODEL = 128
MAX_LEN = 512


def _position_indices(shapes):
    D = shapes[0][1] - 1
    idx_list = []
    for s in shapes:
        t = s[1] - 1
        indices = np.concatenate(
            [np.zeros([1]), np.linspace(1, D, t).astype(np.int32)]
        )
        idx_list.append(indices.astype(np.int64))
    return np.concatenate(idx_list, axis=0).astype(np.int32)


_R_BLK = 304


def _make_fused_gather_broadcast(total: int, batch: int, vocab: int):
    n_steps = -(-total // _R_BLK)

    def body(idx_ref, emb_ref, out_ref):
        cols = jax.lax.broadcasted_iota(jnp.int32, (_R_BLK, vocab), 1)
        onehot = (cols == idx_ref[...]).astype(jnp.float32)
        rows = jnp.dot(
            onehot, emb_ref[...],
            preferred_element_type=jnp.float32,
            precision=jax.lax.Precision.HIGHEST,
        )
        out_ref[...] = jnp.broadcast_to(
            rows[:, None, :], (_R_BLK, batch, D_MODEL)
        )

    return pl.pallas_call(
        body,
        grid=(n_steps,),
        in_specs=[
            pl.BlockSpec((_R_BLK, 1), lambda r: (r, 0)),
            pl.BlockSpec((vocab, D_MODEL), lambda r: (0, 0)),
        ],
        out_specs=pl.BlockSpec((_R_BLK, batch, D_MODEL), lambda r: (r, 0, 0)),
        out_shape=jax.ShapeDtypeStruct((total, batch, D_MODEL), jnp.float32),
    )


def kernel(modal_feat_0, modal_feat_1, modal_feat_2, emb):
    shapes = [modal_feat_0.shape, modal_feat_1.shape, modal_feat_2.shape]
    batch = shapes[0][0]
    idx = _position_indices(shapes)
    total = idx.shape[0]

    n_steps = -(-total // _R_BLK)
    idx_col = np.zeros((n_steps * _R_BLK, 1), np.int32)
    idx_col[:total, 0] = idx

    fused = _make_fused_gather_broadcast(total, batch, emb.shape[0])
    out_t = fused(jnp.asarray(idx_col), emb)
    return jnp.transpose(out_t, (1, 0, 2))
